# baseline (speedup 1.0000x reference)
import jax
import jax.numpy as jnp
from jax.experimental import pallas as pl
from jax.experimental.pallas import tpu as pltpu

_MiB = 1024 * 1024


def _lora_body(x_ref, at_ref, b_ref, o_ref, a_scr):
    @pl.when(pl.program_id(0) == 0)
    def _():
        a_scr[...] = at_ref[...].T.astype(jnp.bfloat16)

    xa = jnp.dot(x_ref[...].astype(jnp.bfloat16), a_scr[...],
                 preferred_element_type=jnp.float32)
    xa = (16.0 * xa).astype(jnp.bfloat16)
    y = jnp.dot(xa, b_ref[...].astype(jnp.bfloat16),
                preferred_element_type=jnp.float32)
    o_ref[...] = y.astype(o_ref.dtype)


def kernel(x, A, B):
    M, K = x.shape
    R, N = B.shape
    assert A.shape == (K, R)
    out_dtype = x.dtype

    tm = min(768, M)
    grid = (pl.cdiv(M, tm),)

    flops = 2 * M * K * R + 2 * M * R * N
    bytes_accessed = (M * K + K * R + R * N + M * N) * 4
    cost = pl.CostEstimate(flops=flops, transcendentals=0,
                           bytes_accessed=bytes_accessed)

    out = pl.pallas_call(
        _lora_body,
        out_shape=jax.ShapeDtypeStruct((M, N), out_dtype),
        grid=grid,
        in_specs=[
            pl.BlockSpec((tm, K), lambda i: (i, 0)),
            pl.BlockSpec((R, K), lambda i: (0, 0),
                         pipeline_mode=pl.Buffered(1)),
            pl.BlockSpec((R, N), lambda i: (0, 0),
                         pipeline_mode=pl.Buffered(1)),
        ],
        out_specs=pl.BlockSpec((tm, N), lambda i: (i, 0)),
        scratch_shapes=[pltpu.VMEM((K, R), jnp.bfloat16)],
        compiler_params=pltpu.CompilerParams(
            dimension_semantics=("arbitrary",),
            vmem_limit_bytes=63 * _MiB,
        ),
        cost_estimate=cost,
    )(x, A.T, B)
    return out


# speedup vs baseline: 1.0106x; 1.0106x over previous
import jax
import jax.numpy as jnp
from jax.experimental import pallas as pl
from jax.experimental.pallas import tpu as pltpu

_MiB = 1024 * 1024


def _lora_body(x_ref, at_ref, b_ref, o_ref, a_scr):
    @pl.when(pl.program_id(0) == 0)
    def _():
        a_scr[...] = at_ref[...].T.astype(jnp.bfloat16)

    xa = jnp.dot(x_ref[...].astype(jnp.bfloat16), a_scr[...],
                 preferred_element_type=jnp.float32)
    xa = (16.0 * xa).astype(jnp.bfloat16)
    y = jnp.dot(xa, b_ref[...].astype(jnp.bfloat16),
                preferred_element_type=jnp.float32)
    o_ref[...] = y.astype(o_ref.dtype)


def kernel(x, A, B):
    M, K = x.shape
    R, N = B.shape
    assert A.shape == (K, R)
    out_dtype = x.dtype

    tm = min(704, M)
    grid = (pl.cdiv(M, tm),)

    flops = 2 * M * K * R + 2 * M * R * N
    bytes_accessed = (M * K + K * R + R * N + M * N) * 4
    cost = pl.CostEstimate(flops=flops, transcendentals=0,
                           bytes_accessed=bytes_accessed)

    out = pl.pallas_call(
        _lora_body,
        out_shape=jax.ShapeDtypeStruct((M, N), out_dtype),
        grid=grid,
        in_specs=[
            pl.BlockSpec((tm, K), lambda i: (i, 0)),
            pl.BlockSpec((R, K), lambda i: (0, 0),
                         pipeline_mode=pl.Buffered(1)),
            pl.BlockSpec((R, N), lambda i: (0, 0),
                         pipeline_mode=pl.Buffered(1)),
        ],
        out_specs=pl.BlockSpec((tm, N), lambda i: (i, 0)),
        scratch_shapes=[pltpu.VMEM((K, R), jnp.bfloat16)],
        compiler_params=pltpu.CompilerParams(
            dimension_semantics=("arbitrary",),
            vmem_limit_bytes=63 * _MiB,
        ),
        cost_estimate=cost,
    )(x, A.T, B)
    return out
